# Initial kernel scaffold; baseline (speedup 1.0000x reference)
import jax
import jax.numpy as jnp
from jax.experimental import pallas as pl


def kernel(x, edge_index, batch, W1, b1, W2, b2, Wg1, bg1, Wg2, bg2, Wc1, bc1, Wc2, bc2):
    raise NotImplementedError("write your pallas kernel here")


# speedup vs baseline: 8.3973x; 8.3973x over previous
import functools

import jax
import jax.numpy as jnp
from jax import lax
from jax.experimental import pallas as pl
from jax.experimental.pallas import tpu as pltpu
from jax.experimental.pallas import tpu_sc as plsc

N = 10000
E = 320000
DIN = 128
DH = 256
DHALF = 128
NG = 16

NC = 2
NS = 16
NPAD = 10240
ROWS_PER_TILE = NPAD // NS
CH = 128
AGG_CHUNKS = 158
E_PAD = NS * AGG_CHUNKS * CH
DEG_CHUNKS = E_PAD // (NC * NS * CH)

_MESH = dict(core_axis_name="c", subcore_axis_name="s", num_cores=NC,
             num_subcores=NS)


def _deg_body(dst_hbm, deg_a, deg_b, idx_v, ones_v, zero_v, hist_sh):
    c = lax.axis_index("c")
    s = lax.axis_index("s")
    w = s * NC + c
    ones16 = jnp.ones((16,), jnp.float32)
    zeros16 = jnp.zeros((16,), jnp.float32)

    def init_row(i, carry):
        for j in range(DHALF // 16):
            ones_v[i, pl.ds(j * 16, 16)] = ones16
            zero_v[i, pl.ds(j * 16, 16)] = zeros16
        return carry

    lax.fori_loop(0, CH, init_row, 0)

    base = s * ROWS_PER_TILE
    for j in range(ROWS_PER_TILE // CH):
        pltpu.sync_copy(zero_v, hist_sh.at[pl.ds(base + j * CH, CH)])
    plsc.subcore_barrier()

    ebase = w * (DEG_CHUNKS * CH)

    def chunk(i, carry):
        eb = pl.multiple_of(ebase + i * CH, CH)
        pltpu.sync_copy(dst_hbm.at[pl.ds(eb, CH)], idx_v)
        pltpu.sync_copy(ones_v, hist_sh.at[idx_v], add=True)
        return carry

    lax.fori_loop(0, DEG_CHUNKS, chunk, 0)
    plsc.subcore_barrier()

    sl = pl.ds(base, ROWS_PER_TILE)

    @pl.when(c == 0)
    def _():
        pltpu.sync_copy(hist_sh.at[sl], deg_a.at[sl])

    @pl.when(c == 1)
    def _():
        pltpu.sync_copy(hist_sh.at[sl], deg_b.at[sl])


def _agg_body(src_hbm, dst_hbm, h_a, h_b, out_a, out_b,
              sidx, didx, rows_v, acc_sh, sem):
    c = lax.axis_index("c")
    s = lax.axis_index("s")
    zeros16 = jnp.zeros((16,), jnp.float32)

    def zrow(i, carry):
        for j in range(DHALF // 16):
            rows_v[i, pl.ds(j * 16, 16)] = zeros16
        return carry

    lax.fori_loop(0, CH, zrow, 0)

    base = s * ROWS_PER_TILE
    for j in range(ROWS_PER_TILE // CH):
        pltpu.sync_copy(rows_v, acc_sh.at[pl.ds(base + j * CH, CH)])
    plsc.subcore_barrier()

    ebase = s * (AGG_CHUNKS * CH)

    def run(h_hbm):
        def chunk(i, carry):
            eb = pl.multiple_of(ebase + i * CH, CH)
            pltpu.sync_copy(src_hbm.at[pl.ds(eb, CH)], sidx)
            pltpu.sync_copy(dst_hbm.at[pl.ds(eb, CH)], didx)
            pltpu.async_copy(h_hbm.at[sidx], rows_v, sem).wait()
            pltpu.sync_copy(rows_v, acc_sh.at[didx], add=True)
            return carry

        lax.fori_loop(0, AGG_CHUNKS, chunk, 0)

    @pl.when(c == 0)
    def _():
        run(h_a)

    @pl.when(c == 1)
    def _():
        run(h_b)

    plsc.subcore_barrier()
    sl = pl.ds(base, ROWS_PER_TILE)

    @pl.when(c == 0)
    def _():
        pltpu.sync_copy(acc_sh.at[sl], out_a.at[sl])

    @pl.when(c == 1)
    def _():
        pltpu.sync_copy(acc_sh.at[sl], out_b.at[sl])


def _deg_call(dst):
    f32 = jnp.float32
    return pl.kernel(
        _deg_body,
        out_type=[jax.ShapeDtypeStruct((NPAD, DHALF), f32),
                  jax.ShapeDtypeStruct((NPAD, DHALF), f32)],
        mesh=plsc.VectorSubcoreMesh(**_MESH),
        scratch_types=[pltpu.VMEM((CH,), jnp.int32),
                       pltpu.VMEM((CH, DHALF), f32),
                       pltpu.VMEM((CH, DHALF), f32),
                       pltpu.VMEM_SHARED((NPAD, DHALF), f32)],
    )(dst)


def _agg_call(src, dst, h_a, h_b):
    f32 = jnp.float32
    return pl.kernel(
        _agg_body,
        out_type=[jax.ShapeDtypeStruct((NPAD, DHALF), f32),
                  jax.ShapeDtypeStruct((NPAD, DHALF), f32)],
        mesh=plsc.VectorSubcoreMesh(**_MESH),
        scratch_types=[pltpu.VMEM((CH,), jnp.int32),
                       pltpu.VMEM((CH,), jnp.int32),
                       pltpu.VMEM((CH, DHALF), f32),
                       pltpu.VMEM_SHARED((NPAD, DHALF), f32),
                       pltpu.SemaphoreType.DMA],
    )(src, dst, h_a, h_b)


def _dinv_of(deg_a, deg_b):
    return lax.rsqrt(deg_a[:, 0] + deg_b[:, 0] + 1.0)


def _mm1_body(x_ref, w_ref, da_ref, db_ref, oa_ref, ob_ref):
    dinv = _dinv_of(da_ref[...], db_ref[...])
    h = jnp.dot(x_ref[...], w_ref[...], preferred_element_type=jnp.float32)
    hs = h * dinv[:, None]
    oa_ref[...] = hs[:, :DHALF]
    ob_ref[...] = hs[:, DHALF:]


def _mm2_body(aa_ref, ab_ref, ha_ref, hb_ref, da_ref, db_ref, b_ref, w_ref,
              oa_ref, ob_ref):
    dinv = _dinv_of(da_ref[...], db_ref[...])
    h1 = jnp.concatenate([aa_ref[...] + ha_ref[...],
                          ab_ref[...] + hb_ref[...]], axis=1)
    h1 = jnp.maximum(h1 * dinv[:, None] + b_ref[...], 0.0)
    h2 = jnp.dot(h1, w_ref[...], preferred_element_type=jnp.float32)
    hs = h2 * dinv[:, None]
    oa_ref[...] = hs[:, :DHALF]
    ob_ref[...] = hs[:, DHALF:]


_BLK = 1000


def _mm1_call(x, w1, deg_a, deg_b):
    f32 = jnp.float32
    sds = jax.ShapeDtypeStruct((N, DHALF), f32)
    return pl.pallas_call(
        _mm1_body,
        grid=(N // _BLK,),
        in_specs=[
            pl.BlockSpec((_BLK, DIN), lambda i: (i, 0)),
            pl.BlockSpec((DIN, DH), lambda i: (0, 0)),
            pl.BlockSpec((_BLK, DHALF), lambda i: (i, 0)),
            pl.BlockSpec((_BLK, DHALF), lambda i: (i, 0)),
        ],
        out_specs=[pl.BlockSpec((_BLK, DHALF), lambda i: (i, 0)),
                   pl.BlockSpec((_BLK, DHALF), lambda i: (i, 0))],
        out_shape=[sds, sds],
    )(x, w1, deg_a, deg_b)


def _mm2_call(agg_a, agg_b, h_a, h_b, deg_a, deg_b, b1, w2):
    f32 = jnp.float32
    sds = jax.ShapeDtypeStruct((N, DHALF), f32)
    return pl.pallas_call(
        _mm2_body,
        grid=(N // _BLK,),
        in_specs=[
            pl.BlockSpec((_BLK, DHALF), lambda i: (i, 0)),
            pl.BlockSpec((_BLK, DHALF), lambda i: (i, 0)),
            pl.BlockSpec((_BLK, DHALF), lambda i: (i, 0)),
            pl.BlockSpec((_BLK, DHALF), lambda i: (i, 0)),
            pl.BlockSpec((_BLK, DHALF), lambda i: (i, 0)),
            pl.BlockSpec((_BLK, DHALF), lambda i: (i, 0)),
            pl.BlockSpec((1, DH), lambda i: (0, 0)),
            pl.BlockSpec((DH, DH), lambda i: (0, 0)),
        ],
        out_specs=[pl.BlockSpec((_BLK, DHALF), lambda i: (i, 0)),
                   pl.BlockSpec((_BLK, DHALF), lambda i: (i, 0))],
        out_shape=[sds, sds],
    )(agg_a, agg_b, h_a, h_b, deg_a, deg_b, b1, w2)


def _final_body(aa_ref, ab_ref, ha_ref, hb_ref, da_ref, db_ref, b2_ref,
                wg1_ref, bg1_ref, wg2_ref, bg2_ref, wc1_ref, bc1_ref,
                wc2_ref, bc2_ref, bat_ref, o_ref):
    f32 = jnp.float32
    dinv = _dinv_of(da_ref[...][:N], db_ref[...][:N])[:, None]
    h0 = jnp.concatenate([aa_ref[...][:N] + ha_ref[...],
                          ab_ref[...][:N] + hb_ref[...]], axis=1)
    h2 = jnp.maximum(h0 * dinv + b2_ref[...], 0.0)
    gh = jnp.maximum(
        jnp.dot(h2, wg1_ref[...], preferred_element_type=f32) + bg1_ref[...],
        0.0)
    gate = jnp.dot(gh, wg2_ref[...], preferred_element_type=f32) + bg2_ref[...]
    bat = bat_ref[...]
    gids = lax.broadcasted_iota(jnp.int32, (N, NG), 1)
    ohb = bat == gids
    ohf = ohb.astype(f32)
    gmax = jnp.max(jnp.where(ohb, gate, -1e30), axis=0)
    gmax_b = jnp.dot(ohf, gmax[:, None], preferred_element_type=f32)
    e = jnp.exp(gate - gmax_b)
    denom = lax.dot_general(ohf, e, (((0,), (0,)), ((), ())),
                            preferred_element_type=f32)
    denom_b = jnp.dot(ohf, denom, preferred_element_type=f32)
    alpha = e / denom_b
    pooled = lax.dot_general(ohf * alpha, h2, (((0,), (0,)), ((), ())),
                             preferred_element_type=f32)
    cls = jnp.maximum(
        jnp.dot(pooled, wc1_ref[...], preferred_element_type=f32)
        + bc1_ref[...], 0.0)
    o_ref[...] = (jnp.dot(cls, wc2_ref[...], preferred_element_type=f32)
                  + bc2_ref[...])


def _final_call(agg_a, agg_b, h_a, h_b, deg_a, deg_b, b2, wg1, bg1, wg2, bg2,
                wc1, bc1, wc2, bc2, bat):
    return pl.pallas_call(
        _final_body,
        out_shape=jax.ShapeDtypeStruct((NG, 2), jnp.float32),
    )(agg_a, agg_b, h_a, h_b, deg_a, deg_b, b2, wg1, bg1, wg2, bg2,
      wc1, bc1, wc2, bc2, bat)


def kernel(x, edge_index, batch, W1, b1, W2, b2, Wg1, bg1, Wg2, bg2,
           Wc1, bc1, Wc2, bc2):
    i32 = jnp.int32
    ei = edge_index.astype(i32)
    npad = E_PAD - E
    src = jnp.concatenate([ei[0], jnp.zeros((npad,), i32)])
    dst = jnp.concatenate([ei[1], jnp.full((npad,), N, i32)])
    bat = batch.astype(i32).reshape(N, 1)

    deg_a, deg_b = _deg_call(dst)
    hs_a, hs_b = _mm1_call(x, W1, deg_a, deg_b)
    agg_a, agg_b = _agg_call(src, dst, hs_a, hs_b)
    hs2_a, hs2_b = _mm2_call(agg_a[:N], agg_b[:N], hs_a, hs_b, deg_a, deg_b,
                             b1.reshape(1, DH), W2)
    agg2_a, agg2_b = _agg_call(src, dst, hs2_a, hs2_b)
    return _final_call(agg2_a, agg2_b, hs2_a, hs2_b, deg_a, deg_b,
                       b2.reshape(1, DH), Wg1, bg1.reshape(1, DHALF), Wg2,
                       bg2.reshape(1, 1), Wc1, bc1.reshape(1, DHALF), Wc2,
                       bc2.reshape(1, 2), bat)


# speedup vs baseline: 9.1262x; 1.0868x over previous
import functools

import jax
import jax.numpy as jnp
from jax import lax
from jax.experimental import pallas as pl
from jax.experimental.pallas import tpu as pltpu
from jax.experimental.pallas import tpu_sc as plsc

N = 10000
E = 320000
DIN = 128
DH = 256
DHALF = 128
NG = 16

NC = 2
NS = 16
NPAD = 10240
ROWS_PER_TILE = NPAD // NS
CH = 128
AGG_CHUNKS = 160
E_PAD = NS * AGG_CHUNKS * CH
ECH = E_PAD // CH
SCH = 32
NSUP = AGG_CHUNKS // SCH
DEG_CHUNKS = ECH // (NC * NS)

_MESH = dict(core_axis_name="c", subcore_axis_name="s", num_cores=NC,
             num_subcores=NS)


def _deg_body(dst_hbm, deg_a, deg_b, dbuf, ones_v, zero_v, hist_sh):
    c = lax.axis_index("c")
    s = lax.axis_index("s")
    w = s * NC + c
    ones16 = jnp.ones((16,), jnp.float32)
    zeros16 = jnp.zeros((16,), jnp.float32)

    def init_row(i, carry):
        for j in range(DHALF // 16):
            ones_v[i, pl.ds(j * 16, 16)] = ones16
            zero_v[i, pl.ds(j * 16, 16)] = zeros16
        return carry

    lax.fori_loop(0, CH, init_row, 0)

    pltpu.sync_copy(dst_hbm.at[pl.ds(w * DEG_CHUNKS, DEG_CHUNKS)], dbuf)

    base = s * ROWS_PER_TILE
    for j in range(ROWS_PER_TILE // CH):
        pltpu.sync_copy(zero_v, hist_sh.at[pl.ds(base + j * CH, CH)])
    plsc.subcore_barrier()

    def chunk(i, carry):
        pltpu.sync_copy(ones_v, hist_sh.at[dbuf.at[i]], add=True)
        return carry

    lax.fori_loop(0, DEG_CHUNKS, chunk, 0)
    plsc.subcore_barrier()

    sl = pl.ds(base, ROWS_PER_TILE)

    @pl.when(c == 0)
    def _():
        pltpu.sync_copy(hist_sh.at[sl], deg_a.at[sl])

    @pl.when(c == 1)
    def _():
        pltpu.sync_copy(hist_sh.at[sl], deg_b.at[sl])


def _agg_body(src_hbm, dst_hbm, h_a, h_b, out_a, out_b,
              sbuf, dbuf, rows0, rows1, acc_sh, sem0, sem1):
    c = lax.axis_index("c")
    s = lax.axis_index("s")
    zeros16 = jnp.zeros((16,), jnp.float32)

    def zrow(i, carry):
        for j in range(DHALF // 16):
            rows0[i, pl.ds(j * 16, 16)] = zeros16
        return carry

    lax.fori_loop(0, CH, zrow, 0)

    base = s * ROWS_PER_TILE
    for j in range(ROWS_PER_TILE // CH):
        pltpu.sync_copy(rows0, acc_sh.at[pl.ds(base + j * CH, CH)])

    plsc.subcore_barrier()

    def run(h_hbm):
        def super_body(j, carry):
            sb = pl.multiple_of(s * AGG_CHUNKS + j * SCH, 8)
            pltpu.sync_copy(src_hbm.at[pl.ds(sb, SCH)], sbuf)
            pltpu.sync_copy(dst_hbm.at[pl.ds(sb, SCH)], dbuf)
            pltpu.async_copy(h_hbm.at[sbuf.at[0]], rows0, sem0)

            def pair(p, cc):
                k = 2 * p
                pltpu.async_copy(h_hbm.at[sbuf.at[k + 1]], rows1, sem1)
                pltpu.make_async_copy(h_hbm.at[sbuf.at[k]], rows0,
                                      sem0).wait()
                pltpu.sync_copy(rows0, acc_sh.at[dbuf.at[k]], add=True)

                @pl.when(k + 2 < SCH)
                def _():
                    pltpu.async_copy(h_hbm.at[sbuf.at[k + 2]], rows0, sem0)

                pltpu.make_async_copy(h_hbm.at[sbuf.at[k + 1]], rows1,
                                      sem1).wait()
                pltpu.sync_copy(rows1, acc_sh.at[dbuf.at[k + 1]], add=True)
                return cc

            lax.fori_loop(0, SCH // 2, pair, 0)
            return carry

        lax.fori_loop(0, NSUP, super_body, 0)

    @pl.when(c == 0)
    def _():
        run(h_a)

    @pl.when(c == 1)
    def _():
        run(h_b)

    plsc.subcore_barrier()
    sl = pl.ds(base, ROWS_PER_TILE)

    @pl.when(c == 0)
    def _():
        pltpu.sync_copy(acc_sh.at[sl], out_a.at[sl])

    @pl.when(c == 1)
    def _():
        pltpu.sync_copy(acc_sh.at[sl], out_b.at[sl])


def _deg_call(dst):
    f32 = jnp.float32
    return pl.kernel(
        _deg_body,
        out_type=[jax.ShapeDtypeStruct((NPAD, DHALF), f32),
                  jax.ShapeDtypeStruct((NPAD, DHALF), f32)],
        mesh=plsc.VectorSubcoreMesh(**_MESH),
        scratch_types=[pltpu.VMEM((DEG_CHUNKS, CH), jnp.int32),
                       pltpu.VMEM((CH, DHALF), f32),
                       pltpu.VMEM((CH, DHALF), f32),
                       pltpu.VMEM_SHARED((NPAD, DHALF), f32)],
    )(dst)


def _agg_call(src, dst, h_a, h_b):
    f32 = jnp.float32
    return pl.kernel(
        _agg_body,
        out_type=[jax.ShapeDtypeStruct((NPAD, DHALF), f32),
                  jax.ShapeDtypeStruct((NPAD, DHALF), f32)],
        mesh=plsc.VectorSubcoreMesh(**_MESH),
        scratch_types=[pltpu.VMEM((SCH, CH), jnp.int32),
                       pltpu.VMEM((SCH, CH), jnp.int32),
                       pltpu.VMEM((CH, DHALF), f32),
                       pltpu.VMEM((CH, DHALF), f32),
                       pltpu.VMEM_SHARED((NPAD, DHALF), f32),
                       pltpu.SemaphoreType.DMA,
                       pltpu.SemaphoreType.DMA],
    )(src, dst, h_a, h_b)


def _dinv_of(deg_a, deg_b):
    return lax.rsqrt(deg_a[:, 0] + deg_b[:, 0] + 1.0)


def _mm1_body(x_ref, w_ref, da_ref, db_ref, oa_ref, ob_ref):
    dinv = _dinv_of(da_ref[...], db_ref[...])
    h = jnp.dot(x_ref[...], w_ref[...], preferred_element_type=jnp.float32)
    hs = h * dinv[:, None]
    oa_ref[...] = hs[:, :DHALF]
    ob_ref[...] = hs[:, DHALF:]


def _mm2_body(aa_ref, ab_ref, ha_ref, hb_ref, da_ref, db_ref, b_ref, w_ref,
              oa_ref, ob_ref):
    dinv = _dinv_of(da_ref[...], db_ref[...])
    h1 = jnp.concatenate([aa_ref[...] + ha_ref[...],
                          ab_ref[...] + hb_ref[...]], axis=1)
    h1 = jnp.maximum(h1 * dinv[:, None] + b_ref[...], 0.0)
    h2 = jnp.dot(h1, w_ref[...], preferred_element_type=jnp.float32)
    hs = h2 * dinv[:, None]
    oa_ref[...] = hs[:, :DHALF]
    ob_ref[...] = hs[:, DHALF:]


_BLK = 1000


def _mm1_call(x, w1, deg_a, deg_b):
    f32 = jnp.float32
    sds = jax.ShapeDtypeStruct((N, DHALF), f32)
    return pl.pallas_call(
        _mm1_body,
        grid=(N // _BLK,),
        in_specs=[
            pl.BlockSpec((_BLK, DIN), lambda i: (i, 0)),
            pl.BlockSpec((DIN, DH), lambda i: (0, 0)),
            pl.BlockSpec((_BLK, DHALF), lambda i: (i, 0)),
            pl.BlockSpec((_BLK, DHALF), lambda i: (i, 0)),
        ],
        out_specs=[pl.BlockSpec((_BLK, DHALF), lambda i: (i, 0)),
                   pl.BlockSpec((_BLK, DHALF), lambda i: (i, 0))],
        out_shape=[sds, sds],
    )(x, w1, deg_a, deg_b)


def _mm2_call(agg_a, agg_b, h_a, h_b, deg_a, deg_b, b1, w2):
    f32 = jnp.float32
    sds = jax.ShapeDtypeStruct((N, DHALF), f32)
    return pl.pallas_call(
        _mm2_body,
        grid=(N // _BLK,),
        in_specs=[
            pl.BlockSpec((_BLK, DHALF), lambda i: (i, 0)),
            pl.BlockSpec((_BLK, DHALF), lambda i: (i, 0)),
            pl.BlockSpec((_BLK, DHALF), lambda i: (i, 0)),
            pl.BlockSpec((_BLK, DHALF), lambda i: (i, 0)),
            pl.BlockSpec((_BLK, DHALF), lambda i: (i, 0)),
            pl.BlockSpec((_BLK, DHALF), lambda i: (i, 0)),
            pl.BlockSpec((1, DH), lambda i: (0, 0)),
            pl.BlockSpec((DH, DH), lambda i: (0, 0)),
        ],
        out_specs=[pl.BlockSpec((_BLK, DHALF), lambda i: (i, 0)),
                   pl.BlockSpec((_BLK, DHALF), lambda i: (i, 0))],
        out_shape=[sds, sds],
    )(agg_a, agg_b, h_a, h_b, deg_a, deg_b, b1, w2)


def _final_body(aa_ref, ab_ref, ha_ref, hb_ref, da_ref, db_ref, b2_ref,
                wg1_ref, bg1_ref, wg2_ref, bg2_ref, wc1_ref, bc1_ref,
                wc2_ref, bc2_ref, bat_ref, o_ref):
    f32 = jnp.float32
    dinv = _dinv_of(da_ref[...][:N], db_ref[...][:N])[:, None]
    h0 = jnp.concatenate([aa_ref[...][:N] + ha_ref[...],
                          ab_ref[...][:N] + hb_ref[...]], axis=1)
    h2 = jnp.maximum(h0 * dinv + b2_ref[...], 0.0)
    gh = jnp.maximum(
        jnp.dot(h2, wg1_ref[...], preferred_element_type=f32) + bg1_ref[...],
        0.0)
    gate = jnp.dot(gh, wg2_ref[...], preferred_element_type=f32) + bg2_ref[...]
    bat = bat_ref[...]
    gids = lax.broadcasted_iota(jnp.int32, (N, NG), 1)
    ohb = bat == gids
    ohf = ohb.astype(f32)
    gmax = jnp.max(jnp.where(ohb, gate, -1e30), axis=0)
    gmax_b = jnp.dot(ohf, gmax[:, None], preferred_element_type=f32)
    e = jnp.exp(gate - gmax_b)
    denom = lax.dot_general(ohf, e, (((0,), (0,)), ((), ())),
                            preferred_element_type=f32)
    denom_b = jnp.dot(ohf, denom, preferred_element_type=f32)
    alpha = e / denom_b
    pooled = lax.dot_general(ohf * alpha, h2, (((0,), (0,)), ((), ())),
                             preferred_element_type=f32)
    cls = jnp.maximum(
        jnp.dot(pooled, wc1_ref[...], preferred_element_type=f32)
        + bc1_ref[...], 0.0)
    o_ref[...] = (jnp.dot(cls, wc2_ref[...], preferred_element_type=f32)
                  + bc2_ref[...])


def _final_call(agg_a, agg_b, h_a, h_b, deg_a, deg_b, b2, wg1, bg1, wg2, bg2,
                wc1, bc1, wc2, bc2, bat):
    return pl.pallas_call(
        _final_body,
        out_shape=jax.ShapeDtypeStruct((NG, 2), jnp.float32),
    )(agg_a, agg_b, h_a, h_b, deg_a, deg_b, b2, wg1, bg1, wg2, bg2,
      wc1, bc1, wc2, bc2, bat)


def kernel(x, edge_index, batch, W1, b1, W2, b2, Wg1, bg1, Wg2, bg2,
           Wc1, bc1, Wc2, bc2):
    i32 = jnp.int32
    ei = edge_index.astype(i32)
    npad = E_PAD - E
    src = jnp.concatenate([ei[0], jnp.zeros((npad,), i32)]).reshape(ECH, CH)
    dst = jnp.concatenate([ei[1],
                           jnp.full((npad,), N, i32)]).reshape(ECH, CH)
    bat = batch.astype(i32).reshape(N, 1)

    deg_a, deg_b = _deg_call(dst)
    hs_a, hs_b = _mm1_call(x, W1, deg_a, deg_b)
    agg_a, agg_b = _agg_call(src, dst, hs_a, hs_b)
    hs2_a, hs2_b = _mm2_call(agg_a[:N], agg_b[:N], hs_a, hs_b, deg_a, deg_b,
                             b1.reshape(1, DH), W2)
    agg2_a, agg2_b = _agg_call(src, dst, hs2_a, hs2_b)
    return _final_call(agg2_a, agg2_b, hs2_a, hs2_b, deg_a, deg_b,
                       b2.reshape(1, DH), Wg1, bg1.reshape(1, DHALF), Wg2,
                       bg2.reshape(1, 1), Wc1, bc1.reshape(1, DHALF), Wc2,
                       bc2.reshape(1, 2), bat)
